# baseline (speedup 1.0000x reference)
import jax
import jax.numpy as jnp
from jax.experimental import pallas as pl
from jax.experimental.pallas import tpu as pltpu

N = 10000
BR = 200

_P = jax.lax.Precision.DEFAULT


def _h0_kernel(x_ref, w0_ref, h0t_ref):
    h0t_ref[...] = jax.lax.dot_general(
        w0_ref[...], x_ref[...], (((0,), (1,)), ((), ())),
        preferred_element_type=jnp.float32, precision=_P,
    )


def _gcn_kernel(a0_ref, a1_ref, h0t_ref, w1_ref, o0_ref, o1_ref,
                gt3_ref, gt_ref):
    i = pl.program_id(0)
    nb = gt3_ref.shape[0]
    half = nb // 2

    @pl.when(i < half)
    def _():
        for a_ref, slot in ((a0_ref, 2 * i), (a1_ref, 2 * i + 1)):
            zt = jax.lax.dot_general(
                h0t_ref[...], a_ref[0], (((1,), (1,)), ((), ())),
                preferred_element_type=jnp.float32, precision=_P,
            )
            zt = jnp.maximum(zt, 0.0)
            gt3_ref[slot] = jax.lax.dot_general(
                w1_ref[...], zt, (((0,), (0,)), ((), ())),
                preferred_element_type=jnp.float32, precision=_P,
            )

    @pl.when(i == half)
    def _():
        gt_ref[...] = jnp.concatenate(
            [gt3_ref[b] for b in range(nb)], axis=-1)

    @pl.when(i >= half)
    def _():
        for a_ref, o_ref in ((a0_ref, o0_ref), (a1_ref, o1_ref)):
            lt = jax.lax.dot_general(
                gt_ref[...], a_ref[0], (((1,), (1,)), ((), ())),
                preferred_element_type=jnp.float32, precision=_P,
            )
            m = jnp.max(lt, axis=0, keepdims=True)
            e = jnp.exp(lt - m)
            o_ref[0] = (e / jnp.sum(e, axis=0, keepdims=True)).T


def kernel(x, a, W0, W1):
    n, f_in = x.shape
    c0 = W0.shape[1]
    c1 = W1.shape[1]
    nb = n // BR
    half = nb // 2

    h0t = pl.pallas_call(
        _h0_kernel,
        out_shape=jax.ShapeDtypeStruct((c0, n), jnp.float32),
    )(x, W0)

    a3 = a.reshape(half, 2 * BR, n)

    o0, o1 = pl.pallas_call(
        _gcn_kernel,
        grid=(2 * half,),
        in_specs=[
            pl.BlockSpec((1, BR, n), lambda i: (i % (n // BR // 2), 0, 0)),
            pl.BlockSpec((1, BR, n), lambda i: (i % (n // BR // 2), 1, 0)),
            pl.BlockSpec((c0, n), lambda i: (0, 0)),
            pl.BlockSpec((c0, c1), lambda i: (0, 0)),
        ],
        out_specs=[
            pl.BlockSpec(
                (1, BR, c1),
                lambda i: (jnp.maximum(i - n // BR // 2, 0), 0, 0)),
            pl.BlockSpec(
                (1, BR, c1),
                lambda i: (jnp.maximum(i - n // BR // 2, 0), 0, 0)),
        ],
        out_shape=[
            jax.ShapeDtypeStruct((half, BR, c1), jnp.float32),
            jax.ShapeDtypeStruct((half, BR, c1), jnp.float32),
        ],
        scratch_shapes=[
            pltpu.VMEM((nb, c1, BR), jnp.float32),
            pltpu.VMEM((c1, n), jnp.float32),
        ],
    )(a3, a3, h0t, W1)
    out = jnp.stack([o0, o1], axis=1).reshape(n, c1)
    return out


# speedup vs baseline: 1.0440x; 1.0440x over previous
import jax
import jax.numpy as jnp
from jax.experimental import pallas as pl
from jax.experimental.pallas import tpu as pltpu

N = 10000
BR = 200

_P = jax.lax.Precision.DEFAULT


def _h0_kernel(x_ref, w0_ref, h0t_ref):
    h0t_ref[...] = jax.lax.dot_general(
        w0_ref[...], x_ref[...], (((0,), (1,)), ((), ())),
        preferred_element_type=jnp.float32, precision=_P,
    )


def _gcn_kernel(a_ref, h0t_ref, w1_ref, out_ref, gt3_ref, gt_ref):
    i = pl.program_id(0)
    nb = gt3_ref.shape[0]

    @pl.when(i < nb)
    def _():
        zt = jax.lax.dot_general(
            h0t_ref[...], a_ref[...], (((1,), (1,)), ((), ())),
            preferred_element_type=jnp.float32, precision=_P,
        )
        zt = jnp.maximum(zt, 0.0)
        gt3_ref[i] = jax.lax.dot_general(
            w1_ref[...], zt, (((0,), (0,)), ((), ())),
            preferred_element_type=jnp.float32, precision=_P,
        )

    @pl.when(i == nb)
    def _():
        gt_ref[...] = jnp.concatenate(
            [gt3_ref[b] for b in range(nb)], axis=-1)

    @pl.when(i >= nb)
    def _():
        lt = jax.lax.dot_general(
            gt_ref[...], a_ref[...], (((1,), (1,)), ((), ())),
            preferred_element_type=jnp.float32, precision=_P,
        )
        m = jnp.max(lt, axis=0, keepdims=True)
        e = jnp.exp(lt - m)
        out_ref[...] = (e / jnp.sum(e, axis=0, keepdims=True)).T


def kernel(x, a, W0, W1):
    n, f_in = x.shape
    c0 = W0.shape[1]
    c1 = W1.shape[1]
    nb = n // BR

    h0t = pl.pallas_call(
        _h0_kernel,
        out_shape=jax.ShapeDtypeStruct((c0, n), jnp.float32),
    )(x, W0)

    out = pl.pallas_call(
        _gcn_kernel,
        grid=(2 * nb,),
        in_specs=[
            pl.BlockSpec((BR, n), lambda i: (i % (n // BR), 0)),
            pl.BlockSpec((c0, n), lambda i: (0, 0)),
            pl.BlockSpec((c0, c1), lambda i: (0, 0)),
        ],
        out_specs=pl.BlockSpec(
            (BR, c1), lambda i: (jnp.maximum(i - n // BR, 0), 0)),
        out_shape=jax.ShapeDtypeStruct((n, c1), jnp.float32),
        scratch_shapes=[
            pltpu.VMEM((nb, c1, BR), jnp.float32),
            pltpu.VMEM((c1, n), jnp.float32),
        ],
    )(a, h0t, W1)
    return out


# speedup vs baseline: 1.0800x; 1.0344x over previous
import jax
import jax.numpy as jnp
from jax.experimental import pallas as pl
from jax.experimental.pallas import tpu as pltpu

N = 10000
BR = 400

_P = jax.lax.Precision.DEFAULT


def _gcn_kernel(x_ref, a_ref, w0_ref, w1_ref, out_ref, h0t_ref,
                gt3_ref, gt_ref):
    i = pl.program_id(0)
    nb = gt3_ref.shape[0]

    @pl.when(i == 0)
    def _():
        h0t_ref[...] = jax.lax.dot_general(
            w0_ref[...], x_ref[...], (((0,), (1,)), ((), ())),
            preferred_element_type=jnp.float32, precision=_P,
        )

    @pl.when(i < nb)
    def _():
        zt = jax.lax.dot_general(
            h0t_ref[...], a_ref[...], (((1,), (1,)), ((), ())),
            preferred_element_type=jnp.float32, precision=_P,
        )
        zt = jnp.maximum(zt, 0.0)
        gt3_ref[i] = jax.lax.dot_general(
            w1_ref[...], zt, (((0,), (0,)), ((), ())),
            preferred_element_type=jnp.float32, precision=_P,
        )

    @pl.when(i == nb)
    def _():
        gt_ref[...] = jnp.concatenate(
            [gt3_ref[b] for b in range(nb)], axis=-1)

    @pl.when(i >= nb)
    def _():
        lt = jax.lax.dot_general(
            gt_ref[...], a_ref[...], (((1,), (1,)), ((), ())),
            preferred_element_type=jnp.float32, precision=_P,
        )
        m = jnp.max(lt, axis=0, keepdims=True)
        e = jnp.exp(lt - m)
        out_ref[...] = (e / jnp.sum(e, axis=0, keepdims=True)).T


def kernel(x, a, W0, W1):
    n, f_in = x.shape
    c0 = W0.shape[1]
    c1 = W1.shape[1]
    nb = n // BR

    out = pl.pallas_call(
        _gcn_kernel,
        grid=(2 * nb,),
        in_specs=[
            pl.BlockSpec((n, f_in), lambda i: (0, 0)),
            pl.BlockSpec((BR, n), lambda i: (i % (n // BR), 0)),
            pl.BlockSpec((f_in, c0), lambda i: (0, 0)),
            pl.BlockSpec((c0, c1), lambda i: (0, 0)),
        ],
        out_specs=pl.BlockSpec(
            (BR, c1), lambda i: (jnp.maximum(i - n // BR, 0), 0)),
        out_shape=jax.ShapeDtypeStruct((n, c1), jnp.float32),
        scratch_shapes=[
            pltpu.VMEM((c0, n), jnp.float32),
            pltpu.VMEM((nb, c1, BR), jnp.float32),
            pltpu.VMEM((c1, n), jnp.float32),
        ],
    )(x, a, W0, W1)
    return out
